# Initial kernel scaffold; baseline (speedup 1.0000x reference)
import functools
import jax
import jax.numpy as jnp
from jax import lax
from jax.experimental import pallas as pl
from jax.experimental.pallas import tpu as pltpu, tpu_sc as plsc

NC = 2
NS = 16
NW = NC * NS


def _build(n_rows, S, H, V):
    C = 128
    R = n_rows // NW
    assert n_rows % NW == 0 and R % C == 0
    n_chunks = R // C

    mesh = plsc.VectorSubcoreMesh(core_axis_name="c", subcore_axis_name="s")

    @functools.partial(
        pl.kernel,
        out_type=jax.ShapeDtypeStruct((n_rows, H), jnp.float32),
        mesh=mesh,
        scratch_types=[
            pltpu.VMEM((2 * S, H), jnp.float32),
            pltpu.VMEM((C,), jnp.int32),
            pltpu.VMEM((C, H), jnp.float32),
            pltpu.SemaphoreType.DMA,
        ],
    )
    def emb_kernel(x_hbm, emb_hbm, pos_hbm, out_hbm, pos2_v, idx_v, buf_v, sem):
        wid = lax.axis_index("s") * NC + lax.axis_index("c")
        base = wid * R
        pltpu.sync_copy(pos_hbm, pos2_v.at[pl.ds(0, S)])
        pltpu.sync_copy(pos_hbm, pos2_v.at[pl.ds(S, S)])

        @pl.loop(0, n_chunks)
        def chunk(j):
            start = base + j * C
            phase = pl.multiple_of(lax.rem(j * C, S), 8)
            pltpu.sync_copy(x_hbm.at[pl.ds(start, C)], idx_v)
            pltpu.sync_copy(pos2_v.at[pl.ds(phase, C)], buf_v)
            pltpu.async_copy(emb_hbm.at[idx_v], buf_v, sem, add=True).wait()
            pltpu.sync_copy(buf_v, out_hbm.at[pl.ds(start, C)])

    return emb_kernel


def kernel(x, emb_table, pos_table):
    B, S = x.shape
    V, H = emb_table.shape
    xf = x.reshape(B * S).astype(jnp.int32)
    out = _build(B * S, S, H, V)(xf, emb_table, pos_table)
    return out.reshape(B, S, H)


# speedup vs baseline: 4.1949x; 4.1949x over previous
import functools
import jax
import jax.numpy as jnp
from jax import lax
from jax.experimental import pallas as pl
from jax.experimental.pallas import tpu as pltpu, tpu_sc as plsc

NC = 2
NS = 16
NW = NC * NS


def _build(n_rows, S, H, V):
    C = 128
    R = n_rows // NW
    assert n_rows % NW == 0 and R % C == 0
    n_chunks = R // C

    mesh = plsc.VectorSubcoreMesh(core_axis_name="c", subcore_axis_name="s")

    @functools.partial(
        pl.kernel,
        out_type=jax.ShapeDtypeStruct((n_rows, H), jnp.float32),
        mesh=mesh,
        scratch_types=[
            pltpu.VMEM((2 * S, H), jnp.float32),
            pltpu.VMEM_SHARED((2 * S, H), jnp.float32),
            pltpu.VMEM((C,), jnp.int32),
            pltpu.VMEM((C, H), jnp.float32),
            pltpu.SemaphoreType.DMA,
        ],
    )
    def emb_kernel(x_hbm, emb_hbm, pos_hbm, out_hbm, pos2_v, pos2_s, idx_v,
                   buf_v, sem):
        sid = lax.axis_index("s")
        wid = sid * NC + lax.axis_index("c")
        base = wid * R

        @pl.when(sid == 0)
        def _():
            pltpu.sync_copy(pos_hbm, pos2_v.at[pl.ds(0, S)])
            pltpu.sync_copy(pos_hbm, pos2_v.at[pl.ds(S, S)])
            pltpu.sync_copy(pos2_v, pos2_s)

        plsc.subcore_barrier()

        @pl.loop(0, n_chunks)
        def chunk(j):
            start = base + j * C
            phase = pl.multiple_of(lax.rem(j * C, S), 8)
            pltpu.sync_copy(x_hbm.at[pl.ds(start, C)], idx_v)
            pltpu.sync_copy(pos2_s.at[pl.ds(phase, C)], buf_v)
            pltpu.async_copy(emb_hbm.at[idx_v], buf_v, sem, add=True).wait()
            pltpu.sync_copy(buf_v, out_hbm.at[pl.ds(start, C)])

    return emb_kernel


def kernel(x, emb_table, pos_table):
    B, S = x.shape
    V, H = emb_table.shape
    xf = x.reshape(B * S).astype(jnp.int32)
    out = _build(B * S, S, H, V)(xf, emb_table, pos_table)
    return out.reshape(B, S, H)


# speedup vs baseline: 5.5236x; 1.3168x over previous
import functools
import jax
import jax.numpy as jnp
from jax import lax
from jax.experimental import pallas as pl
from jax.experimental.pallas import tpu as pltpu, tpu_sc as plsc

NC = 2
NS = 16
NW = NC * NS


def _build(n_rows, S, H, V):
    C = 128
    NBUF = 2
    R = n_rows // NW
    assert n_rows % NW == 0 and R % (C * NBUF) == 0
    n_bodies = R // (C * NBUF)

    mesh = plsc.VectorSubcoreMesh(core_axis_name="c", subcore_axis_name="s")

    @functools.partial(
        pl.kernel,
        out_type=jax.ShapeDtypeStruct((n_rows, H), jnp.float32),
        mesh=mesh,
        scratch_types=[
            pltpu.VMEM_SHARED((2 * S, H), jnp.float32),
            [pltpu.VMEM((C,), jnp.int32) for _ in range(NBUF)],
            [pltpu.VMEM((C, H), jnp.float32) for _ in range(NBUF)],
            [pltpu.SemaphoreType.DMA for _ in range(NBUF)],
            [pltpu.SemaphoreType.DMA for _ in range(NBUF)],
            pltpu.SemaphoreType.DMA,
            [pltpu.SemaphoreType.DMA for _ in range(NBUF)],
        ],
    )
    def emb_kernel(x_hbm, emb_hbm, pos_hbm, out_hbm, pos2_s, idx, buf,
                   sem_ix, sem_in, sem_g, sem_o):
        sid = lax.axis_index("s")
        wid = sid * NC + lax.axis_index("c")
        base = wid * R

        @pl.when(sid == 0)
        def _():
            pltpu.sync_copy(pos_hbm, pos2_s.at[pl.ds(0, S)])
            pltpu.sync_copy(pos_hbm, pos2_s.at[pl.ds(S, S)])

        plsc.subcore_barrier()

        @pl.loop(0, n_bodies)
        def body(jo):
            j0 = jo * NBUF
            scatters = []
            for b in range(NBUF):
                t = j0 + b
                a1 = pltpu.async_copy(x_hbm.at[pl.ds(base + t * C, C)],
                                      idx[b], sem_ix[b])
                phase = pl.multiple_of(lax.rem(t * C, S), 8)
                a2 = pltpu.async_copy(pos2_s.at[pl.ds(phase, C)], buf[b],
                                      sem_in[b])
                a1.wait()
                a2.wait()
                pltpu.async_copy(emb_hbm.at[idx[b]], buf[b], sem_g,
                                 add=True).wait()
                scatters.append(
                    pltpu.async_copy(buf[b],
                                     out_hbm.at[pl.ds(base + t * C, C)],
                                     sem_o[b]))
            for sc in scatters:
                sc.wait()

    return emb_kernel


def kernel(x, emb_table, pos_table):
    B, S = x.shape
    V, H = emb_table.shape
    xf = x.reshape(B * S).astype(jnp.int32)
    out = _build(B * S, S, H, V)(xf, emb_table, pos_table)
    return out.reshape(B, S, H)


# speedup vs baseline: 6.4646x; 1.1704x over previous
import functools
import jax
import jax.numpy as jnp
from jax import lax
from jax.experimental import pallas as pl
from jax.experimental.pallas import tpu as pltpu, tpu_sc as plsc

NC = 2
NS = 16
NW = NC * NS


def _build(n_rows, S, H, V):
    C = 128
    NBUF = 5
    R = n_rows // NW
    assert n_rows % NW == 0 and R % (C * NBUF) == 0
    n_bodies = R // (C * NBUF)

    mesh = plsc.VectorSubcoreMesh(core_axis_name="c", subcore_axis_name="s")

    @functools.partial(
        pl.kernel,
        out_type=jax.ShapeDtypeStruct((n_rows, H), jnp.float32),
        mesh=mesh,
        scratch_types=[
            pltpu.VMEM_SHARED((2 * S, H), jnp.float32),
            [pltpu.VMEM((C,), jnp.int32) for _ in range(NBUF)],
            [pltpu.VMEM((C, H), jnp.float32) for _ in range(NBUF)],
            [pltpu.SemaphoreType.DMA for _ in range(NBUF)],
            [pltpu.SemaphoreType.DMA for _ in range(NBUF)],
            pltpu.SemaphoreType.DMA,
            [pltpu.SemaphoreType.DMA for _ in range(NBUF)],
        ],
    )
    def emb_kernel(x_hbm, emb_hbm, pos_hbm, out_hbm, pos2_s, idx, buf,
                   sem_ix, sem_in, sem_g, sem_o):
        sid = lax.axis_index("s")
        wid = sid * NC + lax.axis_index("c")
        base = wid * R

        @pl.when(sid == 0)
        def _():
            pltpu.sync_copy(pos_hbm, pos2_s.at[pl.ds(0, S)])
            pltpu.sync_copy(pos_hbm, pos2_s.at[pl.ds(S, S)])

        plsc.subcore_barrier()

        @pl.loop(0, n_bodies)
        def body(jo):
            j0 = jo * NBUF
            preps = []
            for b in range(NBUF):
                t = j0 + b
                a1 = pltpu.async_copy(x_hbm.at[pl.ds(base + t * C, C)],
                                      idx[b], sem_ix[b])
                phase = pl.multiple_of(lax.rem(t * C, S), 8)
                a2 = pltpu.async_copy(pos2_s.at[pl.ds(phase, C)], buf[b],
                                      sem_in[b])
                preps.append((a1, a2))
            scatters = []
            for b in range(NBUF):
                a1, a2 = preps[b]
                a1.wait()
                a2.wait()
                pltpu.async_copy(emb_hbm.at[idx[b]], buf[b], sem_g,
                                 add=True).wait()
                scatters.append(
                    pltpu.async_copy(buf[b],
                                     out_hbm.at[pl.ds(base + (j0 + b) * C, C)],
                                     sem_o[b]))
            for sc in scatters:
                sc.wait()

    return emb_kernel


def kernel(x, emb_table, pos_table):
    B, S = x.shape
    V, H = emb_table.shape
    xf = x.reshape(B * S).astype(jnp.int32)
    out = _build(B * S, S, H, V)(xf, emb_table, pos_table)
    return out.reshape(B, S, H)


# speedup vs baseline: 8.3527x; 1.2921x over previous
import functools
import jax
import jax.numpy as jnp
from jax import lax
from jax.experimental import pallas as pl
from jax.experimental.pallas import tpu as pltpu, tpu_sc as plsc

NC = 2
NS = 16
NW = NC * NS


def _build(n_rows, S, H, V):
    C = 128
    NBUF = 5
    R = n_rows // NW
    assert n_rows % NW == 0 and R % (C * NBUF) == 0
    n_bodies = R // (C * NBUF)

    mesh = plsc.VectorSubcoreMesh(core_axis_name="c", subcore_axis_name="s")

    @functools.partial(
        pl.kernel,
        out_type=jax.ShapeDtypeStruct((n_rows, H), jnp.float32),
        mesh=mesh,
        scratch_types=[
            pltpu.VMEM_SHARED((2 * S, H), jnp.float32),
            [pltpu.VMEM((C,), jnp.int32) for _ in range(NBUF)],
            [pltpu.VMEM((C, H), jnp.float32) for _ in range(NBUF)],
            [pltpu.SemaphoreType.DMA for _ in range(NBUF)],
            [pltpu.SemaphoreType.DMA for _ in range(NBUF)],
            [pltpu.SemaphoreType.DMA for _ in range(NBUF)],
            [pltpu.SemaphoreType.DMA for _ in range(NBUF)],
        ],
    )
    def emb_kernel(x_hbm, emb_hbm, pos_hbm, out_hbm, pos2_s, idx, buf,
                   sem_ix, sem_in, sem_g, sem_o):
        sid = lax.axis_index("s")
        wid = sid * NC + lax.axis_index("c")
        base = wid * R

        @pl.when(sid == 0)
        def _():
            pltpu.sync_copy(pos_hbm, pos2_s.at[pl.ds(0, S)])
            pltpu.sync_copy(pos_hbm, pos2_s.at[pl.ds(S, S)])

        plsc.subcore_barrier()

        @pl.loop(0, n_bodies)
        def body(jo):
            j0 = jo * NBUF
            preps = []
            for b in range(NBUF):
                t = j0 + b
                a1 = pltpu.async_copy(x_hbm.at[pl.ds(base + t * C, C)],
                                      idx[b], sem_ix[b])
                phase = pl.multiple_of(lax.rem(t * C, S), 8)
                a2 = pltpu.async_copy(pos2_s.at[pl.ds(phase, C)], buf[b],
                                      sem_in[b])
                preps.append((a1, a2))
            gathers = []
            for b in range(NBUF):
                a1, a2 = preps[b]
                a1.wait()
                a2.wait()
                gathers.append(
                    pltpu.async_copy(emb_hbm.at[idx[b]], buf[b], sem_g[b],
                                     add=True))
            scatters = []
            for b in range(NBUF):
                gathers[b].wait()
                scatters.append(
                    pltpu.async_copy(buf[b],
                                     out_hbm.at[pl.ds(base + (j0 + b) * C, C)],
                                     sem_o[b]))
            for sc in scatters:
                sc.wait()

    return emb_kernel


def kernel(x, emb_table, pos_table):
    B, S = x.shape
    V, H = emb_table.shape
    xf = x.reshape(B * S).astype(jnp.int32)
    out = _build(B * S, S, H, V)(xf, emb_table, pos_table)
    return out.reshape(B, S, H)


# speedup vs baseline: 9.0262x; 1.0806x over previous
import functools
import jax
import jax.numpy as jnp
from jax import lax
from jax.experimental import pallas as pl
from jax.experimental.pallas import tpu as pltpu, tpu_sc as plsc

NC = 2
NS = 16
NW = NC * NS


def _build(n_rows, S, H, V):
    C = 128
    NBUF = 5
    W = 5
    R = n_rows // NW
    assert n_rows % NW == 0 and R % (C * NBUF * W) == 0
    n_bodies = R // (C * NBUF * W)

    mesh = plsc.VectorSubcoreMesh(core_axis_name="c", subcore_axis_name="s")

    @functools.partial(
        pl.kernel,
        out_type=jax.ShapeDtypeStruct((n_rows, H), jnp.float32),
        mesh=mesh,
        scratch_types=[
            pltpu.VMEM_SHARED((2 * S, H), jnp.float32),
            [pltpu.VMEM((C,), jnp.int32) for _ in range(NBUF)],
            [pltpu.VMEM((C, H), jnp.float32) for _ in range(NBUF)],
            [pltpu.SemaphoreType.DMA for _ in range(NBUF)],
            [pltpu.SemaphoreType.DMA for _ in range(NBUF)],
            [pltpu.SemaphoreType.DMA for _ in range(NBUF)],
            [pltpu.SemaphoreType.DMA for _ in range(NBUF)],
        ],
    )
    def emb_kernel(x_hbm, emb_hbm, pos_hbm, out_hbm, pos2_s, idx, buf,
                   sem_ix, sem_in, sem_g, sem_o):
        sid = lax.axis_index("s")
        wid = sid * NC + lax.axis_index("c")
        base = wid * R

        @pl.when(sid == 0)
        def _():
            pltpu.sync_copy(pos_hbm, pos2_s.at[pl.ds(0, S)])
            pltpu.sync_copy(pos_hbm, pos2_s.at[pl.ds(S, S)])

        plsc.subcore_barrier()

        def prep(t, b):
            a1 = pltpu.async_copy(x_hbm.at[pl.ds(base + t * C, C)],
                                  idx[b], sem_ix[b])
            phase = pl.multiple_of(lax.rem(t * C, S), 8)
            a2 = pltpu.async_copy(pos2_s.at[pl.ds(phase, C)], buf[b],
                                  sem_in[b])
            return a1, a2

        @pl.loop(0, n_bodies)
        def body(jo):
            j0 = jo * NBUF * W
            preps = [prep(j0 + b, b) for b in range(NBUF)]
            scatters = None
            for w in range(W):
                t0 = j0 + w * NBUF
                gathers = []
                for b in range(NBUF):
                    a1, a2 = preps[b]
                    a1.wait()
                    a2.wait()
                    gathers.append(
                        pltpu.async_copy(emb_hbm.at[idx[b]], buf[b],
                                         sem_g[b], add=True))
                scatters = []
                for b in range(NBUF):
                    gathers[b].wait()
                    scatters.append(
                        pltpu.async_copy(
                            buf[b], out_hbm.at[pl.ds(base + (t0 + b) * C, C)],
                            sem_o[b]))
                if w < W - 1:
                    preps = []
                    for b in range(NBUF):
                        scatters[b].wait()
                        preps.append(prep(t0 + NBUF + b, b))
            for sc in scatters:
                sc.wait()

    return emb_kernel


def kernel(x, emb_table, pos_table):
    B, S = x.shape
    V, H = emb_table.shape
    xf = x.reshape(B * S).astype(jnp.int32)
    out = _build(B * S, S, H, V)(xf, emb_table, pos_table)
    return out.reshape(B, S, H)
